# baseline (speedup 1.0000x reference)
import functools

import jax
import jax.numpy as jnp
from jax import lax
from jax.experimental import pallas as pl
from jax.experimental.pallas import tpu as pltpu
from jax.experimental.pallas import tpu_sc as plsc

N = 10000
E = 320000
D = 128

NC = 2
NS = 16
NW = NC * NS
BLK = 128
BPT0 = 152
BPT1 = 160 - BPT0
NBLK = NS * (BPT0 + BPT1)
EPAD = NBLK * BLK
NPAD = 10112
RPT = NPAD // NS


@functools.lru_cache(maxsize=None)
def _make_agg(d, with_deg):
    NR = 2
    NI = 4
    out_type = [jax.ShapeDtypeStruct((NC, NPAD, d), jnp.float32)]
    scratch = [
        pltpu.VMEM((NI, BLK), jnp.int32),
        pltpu.VMEM((NI, BLK), jnp.int32),
        pltpu.VMEM((NR, BLK, d), jnp.float32),
        pltpu.VMEM_SHARED((NPAD, d), jnp.float32),
        pltpu.SemaphoreType.DMA,
        pltpu.SemaphoreType.DMA,
        pltpu.SemaphoreType.DMA,
    ]
    if with_deg:
        out_type.append(jax.ShapeDtypeStruct((NW, 1, NPAD), jnp.float32))
        scratch.insert(3, pltpu.VMEM((NPAD,), jnp.float32))

    mesh = plsc.VectorSubcoreMesh(core_axis_name="c", subcore_axis_name="s",
                                  num_cores=NC, num_subcores=NS)

    def body(*refs):
        if with_deg:
            (table, srcp, dstp, zeros, zdeg,
             out_sum, out_deg, idx_s, idx_d, rows, degv, acc,
             isem, gsem, ssem) = refs
        else:
            (table, srcp, dstp, zeros,
             out_sum, idx_s, idx_d, rows, acc, isem, gsem, ssem) = refs
        c = lax.axis_index("c")
        s = lax.axis_index("s")
        wid = c * NS + s

        ones16 = jnp.ones((16,), jnp.float32)
        nblk = lax.select(c == 0, BPT0, BPT1)
        base = lax.select(c == 0, s * BPT0, NS * BPT0 + s * BPT1) * BLK

        def i_start(blk, ib):
            off = base + blk * BLK
            pltpu.async_copy(srcp.at[pl.ds(off, BLK)], idx_s.at[ib], isem)
            pltpu.async_copy(dstp.at[pl.ds(off, BLK)], idx_d.at[ib], isem)

        def i_wait(ib):
            pltpu.make_async_copy(srcp.at[pl.ds(0, BLK)], idx_s.at[ib],
                                  isem).wait()
            pltpu.make_async_copy(dstp.at[pl.ds(0, BLK)], idx_d.at[ib],
                                  isem).wait()

        def g_start(ib, rb):
            pltpu.async_copy(table.at[idx_s.at[ib]], rows.at[rb], gsem)

        def g_wait(rb):
            pltpu.make_async_copy(table.at[idx_s.at[0]], rows.at[rb],
                                  gsem).wait()

        def s_start(ib, rb):
            pltpu.async_copy(rows.at[rb], acc.at[idx_d.at[ib]], ssem,
                             add=True)

        def s_wait(rb):
            pltpu.make_async_copy(rows.at[rb], acc.at[idx_d.at[0]],
                                  ssem).wait()

        pltpu.sync_copy(zeros.at[pl.ds(s * RPT, RPT)],
                        acc.at[pl.ds(s * RPT, RPT)])
        if with_deg:
            pltpu.sync_copy(zdeg, degv)
        plsc.subcore_barrier()

        for p in range(NI - 1):
            i_start(p, p)
        i_wait(0)
        g_start(0, 0)

        @pl.loop(0, nblk, step=NI)
        def _(g):
            for b4 in range(NI):
                blk = g + b4
                b2 = b4 % NR
                g_wait(b2)
                s_start(b4, b2)
                if with_deg:
                    for j in range(BLK // 16):
                        iv = idx_d[b4, pl.ds(j * 16, 16)]
                        plsc.addupdate_scatter(degv, [iv], ones16)

                @pl.when(blk >= 1)
                def _():
                    s_wait(1 - b2)

                @pl.when(blk + 1 < nblk)
                def _():
                    i_wait((b4 + 1) % NI)
                    g_start((b4 + 1) % NI, 1 - b2)

                @pl.when(blk + NI - 1 < nblk)
                def _():
                    i_start(blk + NI - 1, (b4 + NI - 1) % NI)

        s_wait(1)
        plsc.subcore_barrier()
        pltpu.sync_copy(acc.at[pl.ds(s * RPT, RPT)],
                        out_sum.at[c, pl.ds(s * RPT, RPT)])
        if with_deg:
            pltpu.sync_copy(degv, out_deg.at[wid, 0])

    return pl.kernel(body, out_type=out_type, mesh=mesh, scratch_types=scratch,
                     compiler_params=pltpu.CompilerParams(
                         needs_layout_passes=False))


def _psum(sum_ref):
    t = sum_ref[0]
    for i in range(1, sum_ref.shape[0]):
        t = t + sum_ref[i]
    return t[:N]


def _bn_relu(pre, g, be):
    mu = jnp.mean(pre, axis=0, keepdims=True)
    var = jnp.mean(jnp.square(pre - mu), axis=0, keepdims=True)
    return jnp.maximum((pre - mu) * lax.rsqrt(var + 1e-5) * g + be, 0.0)


def _tc1_body(sum_ref, degt_ref, x_ref, wl_ref, bl_ref, wr_ref, g_ref, be_ref,
              h_out, dinv_out):
    s = _psum(sum_ref)
    deg = jnp.sum(degt_ref[...], axis=1, keepdims=True)[:N]
    dinv = 1.0 / jnp.maximum(deg, 1.0)
    pre = (jnp.dot(s * dinv, wl_ref[...], preferred_element_type=jnp.float32)
           + bl_ref[...]
           + jnp.dot(x_ref[...], wr_ref[...],
                     preferred_element_type=jnp.float32))
    h_out[...] = _bn_relu(pre, g_ref[...], be_ref[...])
    dinv_out[...] = dinv


def _tc2_body(sum_ref, dinv_ref, h_ref, wl_ref, bl_ref, wr_ref, g_ref, be_ref,
              h_out):
    s = _psum(sum_ref)
    pre = (jnp.dot(s * dinv_ref[...], wl_ref[...],
                   preferred_element_type=jnp.float32)
           + bl_ref[...]
           + jnp.dot(h_ref[...], wr_ref[...],
                     preferred_element_type=jnp.float32))
    h_out[...] = _bn_relu(pre, g_ref[...], be_ref[...])


def _tc3_body(sum_ref, dinv_ref, h_ref, wl_ref, bl_ref, wr_ref, g_ref, be_ref,
              wc1_ref, bc1_ref, wc2_ref, bc2_ref, wc3_ref, bc3_ref, out_ref):
    s = _psum(sum_ref)
    pre = (jnp.dot(s * dinv_ref[...], wl_ref[...],
                   preferred_element_type=jnp.float32)
           + bl_ref[...]
           + jnp.dot(h_ref[...], wr_ref[...],
                     preferred_element_type=jnp.float32))
    h = _bn_relu(pre, g_ref[...], be_ref[...])
    a = jnp.maximum(jnp.dot(h, wc1_ref[...],
                            preferred_element_type=jnp.float32)
                    + bc1_ref[...], 0.0)
    b = jnp.maximum(jnp.dot(a, wc2_ref[...],
                            preferred_element_type=jnp.float32)
                    + bc2_ref[...], 0.0)
    out_ref[...] = (jnp.dot(b, wc3_ref[...],
                            preferred_element_type=jnp.float32)
                    + bc3_ref[...])


def _tc_call(body, out_shapes, *args):
    return pl.pallas_call(body, out_shape=out_shapes)(*args)


def kernel(x, edge_index, Wl1, bl1, Wr1, g1, be1, Wl2, bl2, Wr2, g2, be2,
           Wl3, bl3, Wr3, g3, be3, Wc1, bc1, Wc2, bc2, Wc3, bc3):
    f32 = jnp.float32
    ei = edge_index.astype(jnp.int32)
    srcp = jnp.concatenate([ei[0], jnp.zeros((EPAD - E,), jnp.int32)])
    dstp = jnp.concatenate([ei[1], jnp.full((EPAD - E,), N, jnp.int32)])
    z128 = jnp.zeros((NPAD, D), f32)
    zdeg = jnp.zeros((NPAD,), f32)

    row = lambda v: v.reshape(1, -1)

    sum1, degp = _make_agg(D, True)(x, srcp, dstp, z128, zdeg)
    degt = degp.reshape(NW, NPAD).T
    h1, dinv = _tc_call(
        _tc1_body,
        [jax.ShapeDtypeStruct((N, D), f32), jax.ShapeDtypeStruct((N, 1), f32)],
        sum1, degt, x, Wl1, row(bl1), Wr1, row(g1), row(be1))

    sum2, = _make_agg(D, False)(h1, srcp, dstp, z128)
    h2 = _tc_call(
        _tc2_body,
        jax.ShapeDtypeStruct((N, D), f32),
        sum2, dinv, h1, Wl2, row(bl2), Wr2, row(g2), row(be2))

    sum3, = _make_agg(D, False)(h2, srcp, dstp, z128)
    out = _tc_call(
        _tc3_body,
        jax.ShapeDtypeStruct((N, 1), f32),
        sum3, dinv, h2, Wl3, row(bl3), Wr3, row(g3), row(be3),
        Wc1, row(bc1), Wc2, row(bc2), Wc3, row(bc3))

    return out.squeeze(-1)


# speedup vs baseline: 1.0125x; 1.0125x over previous
import functools

import jax
import jax.numpy as jnp
from jax import lax
from jax.experimental import pallas as pl
from jax.experimental.pallas import tpu as pltpu
from jax.experimental.pallas import tpu_sc as plsc

N = 10000
E = 320000
D = 128

NC = 2
NS = 16
NW = NC * NS
BLK = 128
BPT0 = 148
BPT1 = 160 - BPT0
NBLK = NS * (BPT0 + BPT1)
EPAD = NBLK * BLK
NPAD = 10112
RPT = NPAD // NS


@functools.lru_cache(maxsize=None)
def _make_agg(d, with_deg):
    NR = 2
    NI = 4
    out_type = [jax.ShapeDtypeStruct((NC, NPAD, d), jnp.float32)]
    scratch = [
        pltpu.VMEM((NI, BLK), jnp.int32),
        pltpu.VMEM((NI, BLK), jnp.int32),
        pltpu.VMEM((NR, BLK, d), jnp.float32),
        pltpu.VMEM_SHARED((NPAD, d), jnp.float32),
        pltpu.SemaphoreType.DMA,
        pltpu.SemaphoreType.DMA,
        pltpu.SemaphoreType.DMA,
    ]
    if with_deg:
        out_type.append(jax.ShapeDtypeStruct((NW, 1, NPAD), jnp.float32))
        scratch.insert(3, pltpu.VMEM((NPAD,), jnp.float32))

    mesh = plsc.VectorSubcoreMesh(core_axis_name="c", subcore_axis_name="s",
                                  num_cores=NC, num_subcores=NS)

    def body(*refs):
        if with_deg:
            (table, srcp, dstp, zeros, zdeg,
             out_sum, out_deg, idx_s, idx_d, rows, degv, acc,
             isem, gsem, ssem) = refs
        else:
            (table, srcp, dstp, zeros,
             out_sum, idx_s, idx_d, rows, acc, isem, gsem, ssem) = refs
        c = lax.axis_index("c")
        s = lax.axis_index("s")
        wid = c * NS + s

        ones16 = jnp.ones((16,), jnp.float32)
        nblk = lax.select(c == 0, BPT0, BPT1)
        base = lax.select(c == 0, s * BPT0, NS * BPT0 + s * BPT1) * BLK

        def i_start(blk, ib):
            off = base + blk * BLK
            pltpu.async_copy(srcp.at[pl.ds(off, BLK)], idx_s.at[ib], isem)
            pltpu.async_copy(dstp.at[pl.ds(off, BLK)], idx_d.at[ib], isem)

        def i_wait(ib):
            pltpu.make_async_copy(srcp.at[pl.ds(0, BLK)], idx_s.at[ib],
                                  isem).wait()
            pltpu.make_async_copy(dstp.at[pl.ds(0, BLK)], idx_d.at[ib],
                                  isem).wait()

        def g_start(ib, rb):
            pltpu.async_copy(table.at[idx_s.at[ib]], rows.at[rb], gsem)

        def g_wait(rb):
            pltpu.make_async_copy(table.at[idx_s.at[0]], rows.at[rb],
                                  gsem).wait()

        def s_start(ib, rb):
            pltpu.async_copy(rows.at[rb], acc.at[idx_d.at[ib]], ssem,
                             add=True)

        def s_wait(rb):
            pltpu.make_async_copy(rows.at[rb], acc.at[idx_d.at[0]],
                                  ssem).wait()

        pltpu.sync_copy(zeros.at[pl.ds(s * RPT, RPT)],
                        acc.at[pl.ds(s * RPT, RPT)])
        if with_deg:
            pltpu.sync_copy(zdeg, degv)
        plsc.subcore_barrier()

        for p in range(NI - 1):
            i_start(p, p)
        i_wait(0)
        g_start(0, 0)

        @pl.loop(0, nblk, step=NI)
        def _(g):
            for b4 in range(NI):
                blk = g + b4
                b2 = b4 % NR
                g_wait(b2)
                s_start(b4, b2)
                if with_deg:
                    for j in range(BLK // 16):
                        iv = idx_d[b4, pl.ds(j * 16, 16)]
                        plsc.addupdate_scatter(degv, [iv], ones16)

                @pl.when(blk >= 1)
                def _():
                    s_wait(1 - b2)

                @pl.when(blk + 1 < nblk)
                def _():
                    i_wait((b4 + 1) % NI)
                    g_start((b4 + 1) % NI, 1 - b2)

                @pl.when(blk + NI - 1 < nblk)
                def _():
                    i_start(blk + NI - 1, (b4 + NI - 1) % NI)

        s_wait(1)
        plsc.subcore_barrier()
        pltpu.sync_copy(acc.at[pl.ds(s * RPT, RPT)],
                        out_sum.at[c, pl.ds(s * RPT, RPT)])
        if with_deg:
            pltpu.sync_copy(degv, out_deg.at[wid, 0])

    return pl.kernel(body, out_type=out_type, mesh=mesh, scratch_types=scratch,
                     compiler_params=pltpu.CompilerParams(
                         needs_layout_passes=False))


def _psum(sum_ref):
    t = sum_ref[0]
    for i in range(1, sum_ref.shape[0]):
        t = t + sum_ref[i]
    return t[:N]


def _bn_relu(pre, g, be):
    mu = jnp.mean(pre, axis=0, keepdims=True)
    var = jnp.mean(jnp.square(pre - mu), axis=0, keepdims=True)
    return jnp.maximum((pre - mu) * lax.rsqrt(var + 1e-5) * g + be, 0.0)


def _tc1_body(sum_ref, degt_ref, x_ref, wl_ref, bl_ref, wr_ref, g_ref, be_ref,
              h_out, dinv_out):
    s = _psum(sum_ref)
    deg = jnp.sum(degt_ref[...], axis=1, keepdims=True)[:N]
    dinv = 1.0 / jnp.maximum(deg, 1.0)
    pre = (jnp.dot(s * dinv, wl_ref[...], preferred_element_type=jnp.float32)
           + bl_ref[...]
           + jnp.dot(x_ref[...], wr_ref[...],
                     preferred_element_type=jnp.float32))
    h_out[...] = _bn_relu(pre, g_ref[...], be_ref[...])
    dinv_out[...] = dinv


def _tc2_body(sum_ref, dinv_ref, h_ref, wl_ref, bl_ref, wr_ref, g_ref, be_ref,
              h_out):
    s = _psum(sum_ref)
    pre = (jnp.dot(s * dinv_ref[...], wl_ref[...],
                   preferred_element_type=jnp.float32)
           + bl_ref[...]
           + jnp.dot(h_ref[...], wr_ref[...],
                     preferred_element_type=jnp.float32))
    h_out[...] = _bn_relu(pre, g_ref[...], be_ref[...])


def _tc3_body(sum_ref, dinv_ref, h_ref, wl_ref, bl_ref, wr_ref, g_ref, be_ref,
              wc1_ref, bc1_ref, wc2_ref, bc2_ref, wc3_ref, bc3_ref, out_ref):
    s = _psum(sum_ref)
    pre = (jnp.dot(s * dinv_ref[...], wl_ref[...],
                   preferred_element_type=jnp.float32)
           + bl_ref[...]
           + jnp.dot(h_ref[...], wr_ref[...],
                     preferred_element_type=jnp.float32))
    h = _bn_relu(pre, g_ref[...], be_ref[...])
    a = jnp.maximum(jnp.dot(h, wc1_ref[...],
                            preferred_element_type=jnp.float32)
                    + bc1_ref[...], 0.0)
    b = jnp.maximum(jnp.dot(a, wc2_ref[...],
                            preferred_element_type=jnp.float32)
                    + bc2_ref[...], 0.0)
    out_ref[...] = (jnp.dot(b, wc3_ref[...],
                            preferred_element_type=jnp.float32)
                    + bc3_ref[...])


def _tc_call(body, out_shapes, *args):
    return pl.pallas_call(body, out_shape=out_shapes)(*args)


def kernel(x, edge_index, Wl1, bl1, Wr1, g1, be1, Wl2, bl2, Wr2, g2, be2,
           Wl3, bl3, Wr3, g3, be3, Wc1, bc1, Wc2, bc2, Wc3, bc3):
    f32 = jnp.float32
    ei = edge_index.astype(jnp.int32)
    srcp = jnp.concatenate([ei[0], jnp.zeros((EPAD - E,), jnp.int32)])
    dstp = jnp.concatenate([ei[1], jnp.full((EPAD - E,), N, jnp.int32)])
    z128 = jnp.zeros((NPAD, D), f32)
    zdeg = jnp.zeros((NPAD,), f32)

    row = lambda v: v.reshape(1, -1)

    sum1, degp = _make_agg(D, True)(x, srcp, dstp, z128, zdeg)
    degt = degp.reshape(NW, NPAD).T
    h1, dinv = _tc_call(
        _tc1_body,
        [jax.ShapeDtypeStruct((N, D), f32), jax.ShapeDtypeStruct((N, 1), f32)],
        sum1, degt, x, Wl1, row(bl1), Wr1, row(g1), row(be1))

    sum2, = _make_agg(D, False)(h1, srcp, dstp, z128)
    h2 = _tc_call(
        _tc2_body,
        jax.ShapeDtypeStruct((N, D), f32),
        sum2, dinv, h1, Wl2, row(bl2), Wr2, row(g2), row(be2))

    sum3, = _make_agg(D, False)(h2, srcp, dstp, z128)
    out = _tc_call(
        _tc3_body,
        jax.ShapeDtypeStruct((N, 1), f32),
        sum3, dinv, h2, Wl3, row(bl3), Wr3, row(g3), row(be3),
        Wc1, row(bc1), Wc2, row(bc2), Wc3, row(bc3))

    return out.squeeze(-1)


# speedup vs baseline: 2.5371x; 2.5059x over previous
import functools

import jax
import jax.numpy as jnp
from jax import lax
from jax.experimental import pallas as pl
from jax.experimental.pallas import tpu as pltpu
from jax.experimental.pallas import tpu_sc as plsc

N = 10000
E = 320000
D = 128

NC = 2
NS = 16
NW = NC * NS
BLK = 128
BPT0 = 80
BPT1 = 160 - BPT0
NBLK = NS * (BPT0 + BPT1)
EPAD = NBLK * BLK
NPAD = 10112
RPT = NPAD // NS


@functools.lru_cache(maxsize=None)
def _make_agg(d, with_deg):
    NR = 2
    NI = 4
    out_type = [jax.ShapeDtypeStruct((NC, NPAD, d), jnp.float32)]
    scratch = [
        pltpu.VMEM((NI, BLK), jnp.int32),
        pltpu.VMEM((NI, BLK), jnp.int32),
        pltpu.VMEM((NR, BLK, d), jnp.float32),
        pltpu.VMEM_SHARED((NPAD, d), jnp.float32),
        pltpu.SemaphoreType.DMA,
        pltpu.SemaphoreType.DMA,
        pltpu.SemaphoreType.DMA,
    ]
    if with_deg:
        out_type.append(jax.ShapeDtypeStruct((NW, 1, NPAD), jnp.float32))
        scratch.insert(3, pltpu.VMEM((NPAD,), jnp.float32))

    mesh = plsc.VectorSubcoreMesh(core_axis_name="c", subcore_axis_name="s",
                                  num_cores=NC, num_subcores=NS)

    def body(*refs):
        if with_deg:
            (table, srcp, dstp, zeros, zdeg,
             out_sum, out_deg, idx_s, idx_d, rows, degv, acc,
             isem, gsem, ssem) = refs
        else:
            (table, srcp, dstp, zeros,
             out_sum, idx_s, idx_d, rows, acc, isem, gsem, ssem) = refs
        c = lax.axis_index("c")
        s = lax.axis_index("s")
        wid = c * NS + s

        ones16 = jnp.ones((16,), jnp.float32)
        nblk = lax.select(c == 0, BPT0, BPT1)
        base = lax.select(c == 0, s * BPT0, NS * BPT0 + s * BPT1) * BLK

        def i_start(blk, ib):
            off = base + blk * BLK
            pltpu.async_copy(srcp.at[pl.ds(off, BLK)], idx_s.at[ib], isem)
            pltpu.async_copy(dstp.at[pl.ds(off, BLK)], idx_d.at[ib], isem)

        def i_wait(ib):
            pltpu.make_async_copy(srcp.at[pl.ds(0, BLK)], idx_s.at[ib],
                                  isem).wait()
            pltpu.make_async_copy(dstp.at[pl.ds(0, BLK)], idx_d.at[ib],
                                  isem).wait()

        def g_start(ib, rb):
            pltpu.async_copy(table.at[idx_s.at[ib]], rows.at[rb], gsem)

        def g_wait(rb):
            pltpu.make_async_copy(table.at[idx_s.at[0]], rows.at[rb],
                                  gsem).wait()

        def s_start(ib, rb):
            pltpu.async_copy(rows.at[rb], acc.at[idx_d.at[ib]], ssem,
                             add=True)

        def s_wait(rb):
            pltpu.make_async_copy(rows.at[rb], acc.at[idx_d.at[0]],
                                  ssem).wait()

        pltpu.sync_copy(zeros.at[pl.ds(s * RPT, RPT)],
                        acc.at[pl.ds(s * RPT, RPT)])
        if with_deg:
            pltpu.sync_copy(zdeg, degv)
        plsc.subcore_barrier()

        for p in range(NI - 1):
            i_start(p, p)
        i_wait(0)
        g_start(0, 0)

        @pl.loop(0, nblk, step=NI)
        def _(g):
            for b4 in range(NI):
                blk = g + b4
                b2 = b4 % NR
                g_wait(b2)
                s_start(b4, b2)
                if with_deg:
                    for j in range(BLK // 16):
                        iv = idx_d[b4, pl.ds(j * 16, 16)]
                        plsc.addupdate_scatter(degv, [iv], ones16)

                @pl.when(blk >= 1)
                def _():
                    s_wait(1 - b2)

                @pl.when(blk + 1 < nblk)
                def _():
                    i_wait((b4 + 1) % NI)
                    g_start((b4 + 1) % NI, 1 - b2)

                @pl.when(blk + NI - 1 < nblk)
                def _():
                    i_start(blk + NI - 1, (b4 + NI - 1) % NI)

        s_wait(1)
        plsc.subcore_barrier()
        pltpu.sync_copy(acc.at[pl.ds(s * RPT, RPT)],
                        out_sum.at[c, pl.ds(s * RPT, RPT)])
        if with_deg:
            pltpu.sync_copy(degv, out_deg.at[wid, 0])

    return pl.kernel(body, out_type=out_type, mesh=mesh, scratch_types=scratch,
                     compiler_params=pltpu.CompilerParams(
                         needs_layout_passes=False))


def _psum(sum_ref):
    t = sum_ref[0]
    for i in range(1, sum_ref.shape[0]):
        t = t + sum_ref[i]
    return t[:N]


def _bn_relu(pre, g, be):
    mu = jnp.mean(pre, axis=0, keepdims=True)
    var = jnp.mean(jnp.square(pre - mu), axis=0, keepdims=True)
    return jnp.maximum((pre - mu) * lax.rsqrt(var + 1e-5) * g + be, 0.0)


def _tc1_body(sum_ref, degt_ref, x_ref, wl_ref, bl_ref, wr_ref, g_ref, be_ref,
              h_out, dinv_out):
    s = _psum(sum_ref)
    deg = jnp.sum(degt_ref[...], axis=1, keepdims=True)[:N]
    dinv = 1.0 / jnp.maximum(deg, 1.0)
    pre = (jnp.dot(s * dinv, wl_ref[...], preferred_element_type=jnp.float32)
           + bl_ref[...]
           + jnp.dot(x_ref[...], wr_ref[...],
                     preferred_element_type=jnp.float32))
    h_out[...] = _bn_relu(pre, g_ref[...], be_ref[...])
    dinv_out[...] = dinv


def _tc2_body(sum_ref, dinv_ref, h_ref, wl_ref, bl_ref, wr_ref, g_ref, be_ref,
              h_out):
    s = _psum(sum_ref)
    pre = (jnp.dot(s * dinv_ref[...], wl_ref[...],
                   preferred_element_type=jnp.float32)
           + bl_ref[...]
           + jnp.dot(h_ref[...], wr_ref[...],
                     preferred_element_type=jnp.float32))
    h_out[...] = _bn_relu(pre, g_ref[...], be_ref[...])


def _tc3_body(sum_ref, dinv_ref, h_ref, wl_ref, bl_ref, wr_ref, g_ref, be_ref,
              wc1_ref, bc1_ref, wc2_ref, bc2_ref, wc3_ref, bc3_ref, out_ref):
    s = _psum(sum_ref)
    pre = (jnp.dot(s * dinv_ref[...], wl_ref[...],
                   preferred_element_type=jnp.float32)
           + bl_ref[...]
           + jnp.dot(h_ref[...], wr_ref[...],
                     preferred_element_type=jnp.float32))
    h = _bn_relu(pre, g_ref[...], be_ref[...])
    a = jnp.maximum(jnp.dot(h, wc1_ref[...],
                            preferred_element_type=jnp.float32)
                    + bc1_ref[...], 0.0)
    b = jnp.maximum(jnp.dot(a, wc2_ref[...],
                            preferred_element_type=jnp.float32)
                    + bc2_ref[...], 0.0)
    out_ref[...] = (jnp.dot(b, wc3_ref[...],
                            preferred_element_type=jnp.float32)
                    + bc3_ref[...])


def _tc_call(body, out_shapes, *args):
    return pl.pallas_call(body, out_shape=out_shapes)(*args)


def kernel(x, edge_index, Wl1, bl1, Wr1, g1, be1, Wl2, bl2, Wr2, g2, be2,
           Wl3, bl3, Wr3, g3, be3, Wc1, bc1, Wc2, bc2, Wc3, bc3):
    f32 = jnp.float32
    ei = edge_index.astype(jnp.int32)
    pad_i = jnp.arange(EPAD - E, dtype=jnp.int32)
    srcp = jnp.concatenate([ei[0], pad_i % N])
    dstp = jnp.concatenate([ei[1], N + pad_i % (NPAD - N)])
    z128 = jnp.zeros((NPAD, D), f32)
    zdeg = jnp.zeros((NPAD,), f32)

    row = lambda v: v.reshape(1, -1)

    sum1, degp = _make_agg(D, True)(x, srcp, dstp, z128, zdeg)
    degt = degp.reshape(NW, NPAD).T
    h1, dinv = _tc_call(
        _tc1_body,
        [jax.ShapeDtypeStruct((N, D), f32), jax.ShapeDtypeStruct((N, 1), f32)],
        sum1, degt, x, Wl1, row(bl1), Wr1, row(g1), row(be1))

    sum2, = _make_agg(D, False)(h1, srcp, dstp, z128)
    h2 = _tc_call(
        _tc2_body,
        jax.ShapeDtypeStruct((N, D), f32),
        sum2, dinv, h1, Wl2, row(bl2), Wr2, row(g2), row(be2))

    sum3, = _make_agg(D, False)(h2, srcp, dstp, z128)
    out = _tc_call(
        _tc3_body,
        jax.ShapeDtypeStruct((N, 1), f32),
        sum3, dinv, h2, Wl3, row(bl3), Wr3, row(g3), row(be3),
        Wc1, row(bc1), Wc2, row(bc2), Wc3, row(bc3))

    return out.squeeze(-1)
